# Initial kernel scaffold; baseline (speedup 1.0000x reference)
import jax
import jax.numpy as jnp
from jax.experimental import pallas as pl


def kernel(ids, mapper):
    raise NotImplementedError("write your pallas kernel here")


# speedup vs baseline: 1.7534x; 1.7534x over previous
import jax
import jax.numpy as jnp
from jax import lax
from jax.experimental import pallas as pl
from jax.experimental.pallas import tpu as pltpu
from jax.experimental.pallas import tpu_sc as plsc

_NC = 2
_NS = 16
_NW = _NC * _NS


def _make_gather(n, per_w):
    mesh = plsc.VectorSubcoreMesh(core_axis_name="c", subcore_axis_name="s")

    def body(mapper_hbm, idx_hbm, out_hbm, idx_v, val_v, sem):
        wid = lax.axis_index("s") * _NC + lax.axis_index("c")
        base = wid * per_w
        pltpu.sync_copy(idx_hbm.at[pl.ds(base, per_w)], idx_v)
        pltpu.async_copy(mapper_hbm.at[idx_v], val_v, sem).wait()
        pltpu.sync_copy(val_v, out_hbm.at[pl.ds(base, per_w)])

    return pl.kernel(
        body,
        out_type=jax.ShapeDtypeStruct((n,), jnp.int32),
        mesh=mesh,
        scratch_types=[
            pltpu.VMEM((per_w,), jnp.int32),
            pltpu.VMEM((per_w,), jnp.int32),
            pltpu.SemaphoreType.DMA,
        ],
    )


def kernel(ids, mapper):
    batch, fields = ids.shape
    n = batch * fields
    assert n % (8 * _NW) == 0
    per_w = n // _NW
    idx32 = ids.astype(jnp.int32).reshape(n)
    mapper32 = mapper.astype(jnp.int32)
    out32 = _make_gather(n, per_w)(mapper32, idx32)
    return out32.reshape(batch, fields).astype(mapper.dtype)


# speedup vs baseline: 6.0095x; 3.4273x over previous
import jax
import jax.numpy as jnp
from jax import lax
from jax.experimental import pallas as pl
from jax.experimental.pallas import tpu as pltpu
from jax.experimental.pallas import tpu_sc as plsc

_NC = 2
_NS = 16
_NW = _NC * _NS


def _make_gather(n, per_w):
    mesh = plsc.VectorSubcoreMesh(core_axis_name="c", subcore_axis_name="s")

    def body(mapper_hbm, idx_hbm, out_hbm, idx_v, val_v, sem):
        wid = lax.axis_index("s") * _NC + lax.axis_index("c")
        base = wid * per_w
        pltpu.sync_copy(idx_hbm.at[pl.ds(base, per_w)], idx_v)
        pltpu.async_copy(mapper_hbm.at[idx_v], val_v, sem).wait()
        pltpu.sync_copy(val_v, out_hbm.at[pl.ds(base, per_w)])

    return pl.kernel(
        body,
        out_type=jax.ShapeDtypeStruct((n,), jnp.uint32),
        mesh=mesh,
        scratch_types=[
            pltpu.VMEM((per_w,), jnp.int32),
            pltpu.VMEM((per_w,), jnp.uint32),
            pltpu.SemaphoreType.DMA,
        ],
    )


def kernel(ids, mapper):
    batch, fields = ids.shape
    n = batch * fields
    assert n % (8 * _NW) == 0
    per_w = n // _NW
    idx32 = ids.T.reshape(n).astype(jnp.int32)
    table = mapper.astype(jnp.uint32)
    out32 = _make_gather(n, per_w)(table, idx32)
    return out32.reshape(fields, batch).T.astype(mapper.dtype)


# speedup vs baseline: 6.0625x; 1.0088x over previous
import jax
import jax.numpy as jnp
from jax import lax
from jax.experimental import pallas as pl
from jax.experimental.pallas import tpu as pltpu
from jax.experimental.pallas import tpu_sc as plsc

_NC = 2
_NS = 16
_NW = _NC * _NS


def _make_gather(n, per_w):
    mesh = plsc.VectorSubcoreMesh(core_axis_name="c", subcore_axis_name="s")

    half = per_w // 2

    def body(mapper_hbm, idx_hbm, out_hbm, idx_v, val_v, sem):
        wid = lax.axis_index("s") * _NC + lax.axis_index("c")
        base = wid * per_w
        pltpu.sync_copy(idx_hbm.at[pl.ds(base, per_w)], idx_v)
        c0 = pltpu.async_copy(
            mapper_hbm.at[idx_v.at[pl.ds(0, half)]],
            val_v.at[pl.ds(0, half)], sem)
        c1 = pltpu.async_copy(
            mapper_hbm.at[idx_v.at[pl.ds(half, half)]],
            val_v.at[pl.ds(half, half)], sem)
        c0.wait()
        c1.wait()
        pltpu.sync_copy(val_v, out_hbm.at[pl.ds(base, per_w)])

    return pl.kernel(
        body,
        out_type=jax.ShapeDtypeStruct((n,), jnp.uint32),
        mesh=mesh,
        scratch_types=[
            pltpu.VMEM((per_w,), jnp.int32),
            pltpu.VMEM((per_w,), jnp.uint32),
            pltpu.SemaphoreType.DMA,
        ],
    )


def kernel(ids, mapper):
    batch, fields = ids.shape
    n = batch * fields
    assert n % (8 * _NW) == 0
    per_w = n // _NW
    idx32 = ids.T.reshape(n).astype(jnp.int32)
    table = mapper.astype(jnp.uint32)
    out32 = _make_gather(n, per_w)(table, idx32)
    return out32.reshape(fields, batch).T.astype(mapper.dtype)
